# Initial kernel scaffold; baseline (speedup 1.0000x reference)
import jax
import jax.numpy as jnp
from jax.experimental import pallas as pl


def kernel(x, W, router_bias):
    raise NotImplementedError("write your pallas kernel here")


# speedup vs baseline: 1.4356x; 1.4356x over previous
import functools

import jax
import jax.numpy as jnp
from jax import lax
from jax.experimental import pallas as pl

TOPK = 8


def _router_body(x_ref, wt_ref, bias_ref, w_out_ref, i_out_ref, *, n_experts):
    tb = x_ref.shape[0]
    logits = jnp.dot(x_ref[...], wt_ref[...],
                     preferred_element_type=jnp.float32)
    scores = jax.nn.sigmoid(logits)
    bal = scores + bias_ref[...]
    iota = lax.broadcasted_iota(jnp.int32, (tb, n_experts), 1)

    work = bal
    neg_inf = jnp.float32(-jnp.inf)
    ws, idxs = [], []
    for _ in range(TOPK):
        m = jnp.max(work, axis=1, keepdims=True)
        idx = jnp.min(jnp.where(work == m, iota, n_experts), axis=1,
                      keepdims=True)
        sel = iota == idx
        w = jnp.max(jnp.where(sel, scores, neg_inf), axis=1, keepdims=True)
        work = jnp.where(sel, neg_inf, work)
        ws.append(w)
        idxs.append(idx)

    wcat = jnp.concatenate(ws, axis=1)
    icat = jnp.concatenate(idxs, axis=1)
    l1 = jnp.maximum(jnp.sum(jnp.abs(wcat), axis=1, keepdims=True), 1e-12)
    w_out_ref[...] = wcat / l1
    i_out_ref[...] = icat


def kernel(x, W, router_bias):
    tokens, hidden = x.shape
    n_experts = W.shape[0]
    tb = 512
    grid = (tokens // tb,)
    wt = W.T
    bias2d = router_bias.reshape(1, n_experts)

    body = functools.partial(_router_body, n_experts=n_experts)
    w_out, i_out = pl.pallas_call(
        body,
        grid=grid,
        in_specs=[
            pl.BlockSpec((tb, hidden), lambda i: (i, 0)),
            pl.BlockSpec((hidden, n_experts), lambda i: (0, 0)),
            pl.BlockSpec((1, n_experts), lambda i: (0, 0)),
        ],
        out_specs=[
            pl.BlockSpec((tb, TOPK), lambda i: (i, 0)),
            pl.BlockSpec((tb, TOPK), lambda i: (i, 0)),
        ],
        out_shape=[
            jax.ShapeDtypeStruct((tokens, TOPK), jnp.float32),
            jax.ShapeDtypeStruct((tokens, TOPK), jnp.int32),
        ],
    )(x, wt, bias2d)
    return (w_out, i_out)


# speedup vs baseline: 1.6692x; 1.1627x over previous
import functools

import jax
import jax.numpy as jnp
from jax import lax
from jax.experimental import pallas as pl

TOPK = 8


def _router_body(x_ref, wt_ref, bias_ref, w_out_ref, i_out_ref, *, n_experts):
    tb = x_ref.shape[0]
    logits = jnp.dot(x_ref[...], wt_ref[...],
                     preferred_element_type=jnp.float32)
    scores = jax.nn.sigmoid(logits)
    bal = scores + bias_ref[...]
    iotaf = lax.broadcasted_iota(jnp.int32, (tb, n_experts), 1).astype(
        jnp.float32)
    combo = iotaf + 0.5 * scores

    work = bal
    neg_inf = jnp.float32(-jnp.inf)
    big = jnp.float32(1e9)
    combs = []
    for _ in range(TOPK):
        m = jnp.max(work, axis=1, keepdims=True)
        combined = jnp.min(jnp.where(work == m, combo, big), axis=1,
                           keepdims=True)
        work = jnp.where(combo == combined, neg_inf, work)
        combs.append(combined)

    ccat = jnp.concatenate(combs, axis=1)
    icat = ccat.astype(jnp.int32)
    wcat = (ccat - icat.astype(jnp.float32)) * 2.0
    l1 = jnp.maximum(jnp.sum(jnp.abs(wcat), axis=1, keepdims=True), 1e-12)
    w_out_ref[...] = wcat / l1
    i_out_ref[...] = icat


def kernel(x, W, router_bias):
    tokens, hidden = x.shape
    n_experts = W.shape[0]
    tb = 512
    grid = (tokens // tb,)
    wt = W.T
    bias2d = router_bias.reshape(1, n_experts)

    body = functools.partial(_router_body, n_experts=n_experts)
    w_out, i_out = pl.pallas_call(
        body,
        grid=grid,
        in_specs=[
            pl.BlockSpec((tb, hidden), lambda i: (i, 0)),
            pl.BlockSpec((hidden, n_experts), lambda i: (0, 0)),
            pl.BlockSpec((1, n_experts), lambda i: (0, 0)),
        ],
        out_specs=[
            pl.BlockSpec((tb, TOPK), lambda i: (i, 0)),
            pl.BlockSpec((tb, TOPK), lambda i: (i, 0)),
        ],
        out_shape=[
            jax.ShapeDtypeStruct((tokens, TOPK), jnp.float32),
            jax.ShapeDtypeStruct((tokens, TOPK), jnp.int32),
        ],
    )(x, wt, bias2d)
    return (w_out, i_out)


# speedup vs baseline: 1.8553x; 1.1115x over previous
import functools

import jax
import jax.numpy as jnp
from jax import lax
from jax.experimental import pallas as pl

TOPK = 8


def _router_body(x_ref, wt_ref, bias_ref, w_out_ref, i_out_ref, *, n_experts):
    tb = x_ref.shape[0]
    logits = jnp.dot(x_ref[...], wt_ref[...],
                     preferred_element_type=jnp.float32)
    scores = jax.nn.sigmoid(logits)
    bal = scores + bias_ref[...]
    iotaf = lax.broadcasted_iota(jnp.int32, (tb, n_experts), 1).astype(
        jnp.float32)
    combo = iotaf + 0.5 * scores

    work = bal
    neg_inf = jnp.float32(-jnp.inf)
    big = jnp.float32(1e9)
    combs = []
    for _ in range(TOPK):
        m = jnp.max(work, axis=1, keepdims=True)
        combined = jnp.min(jnp.where(work == m, combo, big), axis=1,
                           keepdims=True)
        work = jnp.where(combo == combined, neg_inf, work)
        combs.append(combined)

    ccat = jnp.concatenate(combs, axis=1)
    icat = ccat.astype(jnp.int32)
    wcat = (ccat - icat.astype(jnp.float32)) * 2.0
    l1 = jnp.maximum(jnp.sum(jnp.abs(wcat), axis=1, keepdims=True), 1e-12)
    w_out_ref[...] = wcat / l1
    i_out_ref[...] = icat


def kernel(x, W, router_bias):
    tokens, hidden = x.shape
    n_experts = W.shape[0]
    tb = 1024
    grid = (tokens // tb,)
    wt = W.T
    bias2d = router_bias.reshape(1, n_experts)

    body = functools.partial(_router_body, n_experts=n_experts)
    w_out, i_out = pl.pallas_call(
        body,
        grid=grid,
        in_specs=[
            pl.BlockSpec((tb, hidden), lambda i: (i, 0)),
            pl.BlockSpec((hidden, n_experts), lambda i: (0, 0)),
            pl.BlockSpec((1, n_experts), lambda i: (0, 0)),
        ],
        out_specs=[
            pl.BlockSpec((tb, TOPK), lambda i: (i, 0)),
            pl.BlockSpec((tb, TOPK), lambda i: (i, 0)),
        ],
        out_shape=[
            jax.ShapeDtypeStruct((tokens, TOPK), jnp.float32),
            jax.ShapeDtypeStruct((tokens, TOPK), jnp.int32),
        ],
    )(x, wt, bias2d)
    return (w_out, i_out)
